# baseline (speedup 1.0000x reference)
import jax
import jax.numpy as jnp
from jax import lax
from jax.experimental import pallas as pl
from jax.experimental.pallas import tpu as pltpu
from jax.experimental.pallas import tpu_sc as plsc

_B = 16384
_F = 26
_V = 1000000
_D = 13

_INFO = plsc.get_sparse_core_info()
_NC = _INFO.num_cores
_NS = _INFO.num_subcores
_NW = _NC * _NS
_L = 16

_ROWS = _B // _NW
_NGATH = _ROWS * _F
_GCH = 128
_NG = _NGATH // _GCH
_RING = 8


def _body(idx_hbm, dense_hbm, wb_hbm, table_hbm, out_hbm,
          idx_v, dense_v, wb_v, flat_v, gath_v, acc_v, sem):
    c = lax.axis_index("c")
    s = lax.axis_index("s")
    wid = s * _NC + c
    base = wid * _ROWS

    pltpu.sync_copy(idx_hbm.at[pl.ds(base * _F, _ROWS * _F)], idx_v)
    pltpu.sync_copy(dense_hbm.at[pl.ds(base * _D, _ROWS * _D)], dense_v)
    pltpu.sync_copy(wb_hbm, wb_v)

    iota = lax.iota(jnp.int32, _L)
    iota_f = iota * _F
    nj = _ROWS // _L

    @pl.loop(0, _F)
    def _flat(f):
        off = f * _V
        for j in range(nj):
            pos = iota_f + (j * _L * _F + f)
            vals = plsc.load_gather(idx_v, [pos])
            flat_v[pl.ds(f * _ROWS + j * _L, _L)] = vals + off

    def _fire(g):
        pltpu.async_copy(table_hbm.at[flat_v.at[pl.ds(g * _GCH, _GCH)]],
                         gath_v.at[pl.ds(g * _GCH, _GCH)], sem)

    def _drain(g):
        pltpu.make_async_copy(table_hbm.at[flat_v.at[pl.ds(g * _GCH, _GCH)]],
                              gath_v.at[pl.ds(g * _GCH, _GCH)], sem).wait()

    for g in range(_RING):
        _fire(g)

    @pl.loop(0, _NG - _RING)
    def _ring(g):
        _drain(g)
        _fire(g + _RING)

    for g in range(_NG - _RING, _NG):
        _drain(g)

    ws = [plsc.load_gather(wb_v, [jnp.full((_L,), k, jnp.int32)])
          for k in range(_F + _D + 1)]
    iota_d = iota * _D

    @pl.loop(0, nj)
    def _acc(j):
        acc = ws[_F + _D]
        for d in range(_D):
            dv = plsc.load_gather(dense_v, [iota_d + (j * _L * _D + d)])
            acc = acc + dv * ws[_F + d]
        for f in range(_F):
            gv = gath_v[pl.ds(f * _ROWS + j * _L, _L)]
            acc = acc + gv * ws[f]
        acc_v[pl.ds(j * _L, _L)] = acc

    pltpu.sync_copy(acc_v, out_hbm.at[pl.ds(base, _ROWS)])


@jax.jit
def _run(sparse_idx_flat, dense_flat, table_flat, wb):
    mesh = plsc.VectorSubcoreMesh(core_axis_name="c", subcore_axis_name="s")
    return pl.kernel(
        _body,
        out_type=jax.ShapeDtypeStruct((_B,), jnp.float32),
        mesh=mesh,
        compiler_params=pltpu.CompilerParams(needs_layout_passes=False),
        scratch_types=[
            pltpu.VMEM((_ROWS * _F,), jnp.int32),
            pltpu.VMEM((_ROWS * _D,), jnp.float32),
            pltpu.VMEM((_F + _D + 1,), jnp.float32),
            pltpu.VMEM((_NGATH,), jnp.int32),
            pltpu.VMEM((_NGATH,), jnp.float32),
            pltpu.VMEM((_ROWS,), jnp.float32),
            pltpu.SemaphoreType.DMA,
        ],
    )(sparse_idx_flat, dense_flat, wb, table_flat)


def kernel(sparse_idx, dense, emb_tables, fc_w, fc_b):
    table_flat = emb_tables.reshape(_F * _V)
    wb = jnp.concatenate([fc_w[:, 0], fc_b])
    out = _run(sparse_idx.reshape(_B * _F), dense.reshape(_B * _D),
               table_flat, wb)
    return out.reshape(_B, 1)


# speedup vs baseline: 1.0032x; 1.0032x over previous
import jax
import jax.numpy as jnp
from jax import lax
from jax.experimental import pallas as pl
from jax.experimental.pallas import tpu as pltpu
from jax.experimental.pallas import tpu_sc as plsc

_B = 16384
_F = 26
_V = 1000000
_D = 13

_INFO = plsc.get_sparse_core_info()
_NC = _INFO.num_cores
_NS = _INFO.num_subcores
_NW = _NC * _NS
_L = 16

_ROWS = _B // _NW
_NGATH = _ROWS * _F
_GCH = 128
_NG = _NGATH // _GCH
_RING = 8


def _body(idx_hbm, dense_hbm, wb_hbm, table_hbm, out_hbm,
          idx_v, dense_v, wb_v, flat_v, gath_v, acc_v, sem):
    c = lax.axis_index("c")
    s = lax.axis_index("s")
    wid = s * _NC + c
    base = wid * _ROWS

    pltpu.sync_copy(idx_hbm.at[pl.ds(base * _F, _ROWS * _F)], idx_v)
    pltpu.sync_copy(dense_hbm.at[pl.ds(base * _D, _ROWS * _D)], dense_v)
    pltpu.sync_copy(wb_hbm, wb_v)

    iota = lax.iota(jnp.int32, _L)
    iota_f = iota * _F
    nj = _ROWS // _L

    @pl.loop(0, _F)
    def _flat(f):
        off = f * _V
        for j in range(nj):
            pos = iota_f + (j * _L * _F + f)
            vals = plsc.load_gather(idx_v, [pos])
            flat_v[pl.ds(f * _ROWS + j * _L, _L)] = vals + off

    pltpu.async_copy(table_hbm.at[flat_v], gath_v, sem).wait()

    ws = [plsc.load_gather(wb_v, [jnp.full((_L,), k, jnp.int32)])
          for k in range(_F + _D + 1)]
    iota_d = iota * _D

    @pl.loop(0, nj)
    def _acc(j):
        acc = ws[_F + _D]
        for d in range(_D):
            dv = plsc.load_gather(dense_v, [iota_d + (j * _L * _D + d)])
            acc = acc + dv * ws[_F + d]
        for f in range(_F):
            gv = gath_v[pl.ds(f * _ROWS + j * _L, _L)]
            acc = acc + gv * ws[f]
        acc_v[pl.ds(j * _L, _L)] = acc

    pltpu.sync_copy(acc_v, out_hbm.at[pl.ds(base, _ROWS)])


@jax.jit
def _run(sparse_idx_flat, dense_flat, table_flat, wb):
    mesh = plsc.VectorSubcoreMesh(core_axis_name="c", subcore_axis_name="s")
    return pl.kernel(
        _body,
        out_type=jax.ShapeDtypeStruct((_B,), jnp.float32),
        mesh=mesh,
        compiler_params=pltpu.CompilerParams(needs_layout_passes=False),
        scratch_types=[
            pltpu.VMEM((_ROWS * _F,), jnp.int32),
            pltpu.VMEM((_ROWS * _D,), jnp.float32),
            pltpu.VMEM((_F + _D + 1,), jnp.float32),
            pltpu.VMEM((_NGATH,), jnp.int32),
            pltpu.VMEM((_NGATH,), jnp.float32),
            pltpu.VMEM((_ROWS,), jnp.float32),
            pltpu.SemaphoreType.DMA,
        ],
    )(sparse_idx_flat, dense_flat, wb, table_flat)


def kernel(sparse_idx, dense, emb_tables, fc_w, fc_b):
    table_flat = emb_tables.reshape(_F * _V)
    wb = jnp.concatenate([fc_w[:, 0], fc_b])
    out = _run(sparse_idx.reshape(_B * _F), dense.reshape(_B * _D),
               table_flat, wb)
    return out.reshape(_B, 1)
